# baseline (speedup 1.0000x reference)
import jax
import jax.numpy as jnp
from jax import lax
from jax.experimental import pallas as pl
from jax.experimental.pallas import tpu as pltpu
from jax.experimental.pallas import tpu_sc as plsc

EPS = 1e-7
HID = 128

NTILES = 16
K = 128
EPT = 37888
EP = NTILES * EPT
ZROWS = 8


def _capb(seg):
    nseg = EPT // seg
    nbmax = (seg + K + K - 1) // K
    return nseg * nbmax + nbmax + 2


def _sc_aggregate_write(nsrc, nranges, rng, seg, nbuf=2):
    ar = rng + 2 * K
    nseg = EPT // seg
    nbmax = (seg + K + K - 1) // K
    capb = _capb(seg)
    capw = (nbmax + 2) * 2 * K
    zpt = ar // NTILES
    opt = rng // NTILES
    ngrp = (nbmax + nbuf - 1) // nbuf

    def body(t_hbm, src_hbm, dst_hbm, out_hbm, lists_hbm, counts_hbm,
             accum, segsrc, segdst, cgd, zbuf, vecbuf, semf, *rest):
        rbufs = rest[:nbuf]
        dslots = rest[nbuf:2 * nbuf]
        sems = rest[2 * nbuf:]
        c = lax.axis_index("c")
        s = lax.axis_index("s")
        bias = c * nsrc
        lanes = lax.iota(jnp.int32, 16)

        def zrow(i, _):
            def zcol(j, _):
                zbuf[i, pl.ds(j * 16, 16)] = jnp.zeros((16,), jnp.float32)
                return 0
            return lax.fori_loop(0, HID // 16, zcol, 0)
        lax.fori_loop(0, ZROWS, zrow, 0)

        cnts = jnp.zeros((16,), jnp.int32)
        for r in range(nranges):
            lo = r * rng
            lbase = (s * nranges + r) * (capb * 2 * K)

            def zacc(z, _):
                pltpu.sync_copy(zbuf, accum.at[pl.ds(s * zpt + z * ZROWS, ZROWS)])
                return 0
            lax.fori_loop(0, zpt // ZROWS, zacc, 0)
            plsc.subcore_barrier()

            def seg_body(sg, blk):
                base = s * EPT + sg * seg
                pltpu.sync_copy(src_hbm.at[pl.ds(base, seg)], segsrc)
                pltpu.sync_copy(dst_hbm.at[pl.ds(base, seg)], segdst)

                def compact(i, n16):
                    src16 = segsrc[pl.ds(i * 16, 16)]
                    dst16 = segdst[pl.ds(i * 16, 16)]
                    msk = (dst16 >= lo) & (dst16 < lo + rng)
                    mi = msk.astype(jnp.int32)
                    pos = n16 + plsc.cumsum(mi) - mi
                    col = pos & (K - 1)
                    fg = lax.shift_left(pos - col, 1) + col
                    plsc.store_scatter(cgd, [fg], src16 + bias, mask=msk)
                    plsc.store_scatter(cgd, [fg + K], dst16 - lo, mask=msk)
                    return n16 + plsc.all_reduce_population_count(msk)
                n16 = lax.fori_loop(0, seg // 16, compact,
                                    jnp.zeros((16,), jnp.int32))
                for t in range(K // 16):
                    pos = n16 + lanes + t * 16
                    col = pos & (K - 1)
                    fg = lax.shift_left(pos - col, 1) + col
                    plsc.store_scatter(cgd, [fg], bias + col)
                    plsc.store_scatter(cgd, [fg + K], rng + col)
                n_sc = jnp.max(n16)
                nb = lax.shift_right_logical(n_sc + (K - 1), 7)

                @pl.when(c == 0)
                def _():
                    pltpu.async_copy(
                        cgd, lists_hbm.at[pl.ds(lbase + blk * (2 * K), capw)],
                        semf)

                for j in range(nbuf):
                    @pl.when(j < nb)
                    def _():
                        pltpu.async_copy(
                            t_hbm.at[cgd.at[pl.ds(j * (2 * K), K)]], rbufs[j], sems[j])

                def grp(g, _):
                    for j in range(nbuf):
                        b = g * nbuf + j

                        @pl.when(b < nb)
                        def _():
                            pltpu.make_async_copy(
                                t_hbm.at[cgd.at[pl.ds(b * (2 * K), K)]],
                                rbufs[j], sems[j]).wait()
                            for t2 in range(K // 16):
                                dslots[j][pl.ds(t2 * 16, 16)] = (
                                    cgd[pl.ds(b * (2 * K) + K + t2 * 16, 16)])
                            pltpu.sync_copy(rbufs[j], accum.at[dslots[j]], add=True)

                            @pl.when(b + nbuf < nb)
                            def _():
                                pltpu.async_copy(
                                    t_hbm.at[cgd.at[pl.ds((b + nbuf) * (2 * K), K)]],
                                    rbufs[j], sems[j])
                    return 0
                lax.fori_loop(0, ngrp, grp, 0)

                @pl.when(c == 0)
                def _():
                    pltpu.make_async_copy(
                        cgd, lists_hbm.at[pl.ds(lbase + blk * (2 * K), capw)],
                        semf).wait()
                return blk + nb
            blk_total = lax.fori_loop(0, nseg, seg_body, jnp.int32(0))
            cnts = jnp.where(lanes == r, blk_total, cnts)
            plsc.subcore_barrier()

            pltpu.sync_copy(accum.at[pl.ds(s * opt, opt)],
                            out_hbm.at[pl.ds((c * nranges + r) * rng + s * opt, opt)])
            plsc.subcore_barrier()

        @pl.when(c == 0)
        def _():
            vecbuf[pl.ds(0, 16)] = cnts
            pltpu.sync_copy(vecbuf, counts_hbm.at[pl.ds(s * 16, 16)])

    mesh = plsc.VectorSubcoreMesh(core_axis_name="c", subcore_axis_name="s")
    return pl.kernel(
        body,
        out_type=[jax.ShapeDtypeStruct((2 * nranges * rng, HID), jnp.float32),
                  jax.ShapeDtypeStruct((NTILES * nranges * capb * 2 * K,), jnp.int32),
                  jax.ShapeDtypeStruct((NTILES * 16,), jnp.int32)],
        mesh=mesh,
        compiler_params=pltpu.CompilerParams(needs_layout_passes=False),
        scratch_types=(
            [pltpu.VMEM_SHARED((ar, HID), jnp.float32),
             pltpu.VMEM((seg,), jnp.int32),
             pltpu.VMEM((seg,), jnp.int32),
             pltpu.VMEM(((nbmax + 2) * 2 * K,), jnp.int32),
             pltpu.VMEM((ZROWS, HID), jnp.float32),
             pltpu.VMEM((16,), jnp.int32),
             pltpu.SemaphoreType.DMA]
            + [pltpu.VMEM((K, HID), jnp.float32) for _ in range(nbuf)]
            + [pltpu.VMEM((K,), jnp.int32) for _ in range(nbuf)]
            + [pltpu.SemaphoreType.DMA for _ in range(nbuf)]
        ),
    )


def _sc_aggregate_read(nsrc, nranges, rng, seg, nbuf):
    ar = rng + 2 * K
    nseg = EPT // seg
    nbmax = (seg + K + K - 1) // K
    capb = _capb(seg)
    maxblk = nseg * nbmax
    ni = 2 * nbuf
    ngrps = (maxblk + ni - 1) // ni
    zpt = ar // NTILES
    opt = rng // NTILES

    def body(t_hbm, lists_hbm, counts_hbm, out_hbm,
             accum, cntbuf, zbuf, *rest):
        islots = rest[:2 * nbuf]
        dslots = rest[2 * nbuf:3 * nbuf]
        rbufs = rest[3 * nbuf:4 * nbuf]
        semi = rest[4 * nbuf:6 * nbuf]
        semg = rest[6 * nbuf:7 * nbuf]
        c = lax.axis_index("c")
        s = lax.axis_index("s")
        bias = c * nsrc
        lanes = lax.iota(jnp.int32, 16)

        def zrow(i, _):
            def zcol(j, _):
                zbuf[i, pl.ds(j * 16, 16)] = jnp.zeros((16,), jnp.float32)
                return 0
            return lax.fori_loop(0, HID // 16, zcol, 0)
        lax.fori_loop(0, ZROWS, zrow, 0)

        pltpu.sync_copy(counts_hbm.at[pl.ds(s * 16, 16)], cntbuf)
        cnt16 = cntbuf[pl.ds(0, 16)]

        for r in range(nranges):
            lbase = (s * nranges + r) * (capb * 2 * K)
            nbt = jnp.max(jnp.where(lanes == r, cnt16, 0))

            def zacc(z, _):
                pltpu.sync_copy(zbuf, accum.at[pl.ds(s * zpt + z * ZROWS, ZROWS)])
                return 0
            lax.fori_loop(0, zpt // ZROWS, zacc, 0)
            plsc.subcore_barrier()

            def fire_idx(b, jj):
                pltpu.async_copy(
                    lists_hbm.at[pl.ds(lbase + b * (2 * K), 2 * K)],
                    islots[jj], semi[jj])

            for jj in range(ni):
                @pl.when(jj < nbt)
                def _():
                    fire_idx(jj, jj)

            def grp(g, _):
                for half in range(2):
                    for j in range(nbuf):
                        jj = half * nbuf + j
                        b = g * ni + jj

                        @pl.when(b < nbt)
                        def _():
                            pltpu.make_async_copy(
                                lists_hbm.at[pl.ds(lbase + b * (2 * K), 2 * K)],
                                islots[jj], semi[jj]).wait()
                            for t2 in range(K // 16):
                                islots[jj][pl.ds(t2 * 16, 16)] = (
                                    islots[jj][pl.ds(t2 * 16, 16)] + bias)
                            pltpu.async_copy(
                                t_hbm.at[islots[jj].at[pl.ds(0, K)]],
                                rbufs[j], semg[j])
                    for j in range(nbuf):
                        jj = half * nbuf + j
                        b = g * ni + jj

                        @pl.when(b < nbt)
                        def _():
                            for t2 in range(K // 16):
                                dslots[j][pl.ds(t2 * 16, 16)] = (
                                    islots[jj][pl.ds(K + t2 * 16, 16)])
                            pltpu.make_async_copy(
                                t_hbm.at[islots[jj].at[pl.ds(0, K)]],
                                rbufs[j], semg[j]).wait()
                            pltpu.sync_copy(rbufs[j], accum.at[dslots[j]], add=True)

                            @pl.when(b + ni < nbt)
                            def _():
                                fire_idx(b + ni, jj)
                return 0
            lax.fori_loop(0, ngrps, grp, 0)
            plsc.subcore_barrier()

            pltpu.sync_copy(accum.at[pl.ds(s * opt, opt)],
                            out_hbm.at[pl.ds((c * nranges + r) * rng + s * opt, opt)])
            plsc.subcore_barrier()

    mesh = plsc.VectorSubcoreMesh(core_axis_name="c", subcore_axis_name="s")
    return pl.kernel(
        body,
        out_type=jax.ShapeDtypeStruct((2 * nranges * rng, HID), jnp.float32),
        mesh=mesh,
        compiler_params=pltpu.CompilerParams(needs_layout_passes=False),
        scratch_types=(
            [pltpu.VMEM_SHARED((ar, HID), jnp.float32),
             pltpu.VMEM((16,), jnp.int32),
             pltpu.VMEM((ZROWS, HID), jnp.float32)]
            + [pltpu.VMEM((2 * K,), jnp.int32) for _ in range(2 * nbuf)]
            + [pltpu.VMEM((K,), jnp.int32) for _ in range(nbuf)]
            + [pltpu.VMEM((K, HID), jnp.float32) for _ in range(nbuf)]
            + [pltpu.SemaphoreType.DMA for _ in range(2 * nbuf)]
            + [pltpu.SemaphoreType.DMA for _ in range(nbuf)]
        ),
    )


def _pad_edges(src, dst, ndst):
    npad = EP - src.shape[0]
    fill = jnp.arange(npad, dtype=jnp.int32) % K
    gsrc = jnp.concatenate([src.astype(jnp.int32), fill])
    dpad = jnp.concatenate([dst.astype(jnp.int32), ndst + fill])
    return gsrc, dpad


def _mlp_body(g_ref, w1_ref, b1_ref, w2_ref, b2_ref, o_ref):
    g = g_ref[...]
    h = jnp.maximum(
        jax.lax.dot_general(g, w1_ref[...], (((1,), (0,)), ((), ())),
                            preferred_element_type=jnp.float32) + b1_ref[...], 0.0)
    o_ref[...] = jax.lax.dot_general(h, w2_ref[...], (((1,), (0,)), ((), ())),
                                     preferred_element_type=jnp.float32) + b2_ref[...]


BLK = 1024


def _mlp(g, W1, b1, W2, b2):
    n, k = g.shape
    k2 = W1.shape[1]
    ko = W2.shape[1]
    return pl.pallas_call(
        _mlp_body,
        grid=(n // BLK,),
        in_specs=[
            pl.BlockSpec((BLK, k), lambda i: (i, 0)),
            pl.BlockSpec((k, k2), lambda i: (0, 0)),
            pl.BlockSpec((1, k2), lambda i: (0, 0)),
            pl.BlockSpec((k2, ko), lambda i: (0, 0)),
            pl.BlockSpec((1, ko), lambda i: (0, 0)),
        ],
        out_specs=pl.BlockSpec((BLK, ko), lambda i: (i, 0)),
        out_shape=jax.ShapeDtypeStruct((n, ko), jnp.float32),
    )(g, W1, b1.reshape(1, -1), W2, b2.reshape(1, -1))


def _enc(xp, We, be, bp):
    n = xp.shape[0]
    fin = xp.shape[1]
    h2 = We.shape[1]

    def body(x_ref, we_ref, be_ref, bp_ref, o_ref):
        a = jax.lax.dot_general(x_ref[...], we_ref[...], (((1,), (0,)), ((), ())),
                                preferred_element_type=jnp.float32) + be_ref[...]
        b = jnp.broadcast_to(bp_ref[...], (BLK, h2))
        o_ref[...] = jnp.maximum(jnp.concatenate([a, b], axis=1), 0.0)

    return pl.pallas_call(
        body,
        grid=(n // BLK,),
        in_specs=[
            pl.BlockSpec((BLK, fin), lambda i: (i, 0)),
            pl.BlockSpec((fin, h2), lambda i: (0, 0)),
            pl.BlockSpec((1, h2), lambda i: (0, 0)),
            pl.BlockSpec((1, h2), lambda i: (0, 0)),
        ],
        out_specs=pl.BlockSpec((BLK, 2 * h2), lambda i: (i, 0)),
        out_shape=jax.ShapeDtypeStruct((n, 2 * h2), jnp.float32),
    )(xp, We, be.reshape(1, -1), bp.reshape(1, -1))


def _prep(x, M):
    n = x.shape[0]
    nblk = n // BLK

    def body(x_ref, m_ref, t_ref):
        i = pl.program_id(0)
        y = jnp.maximum(x_ref[...], 0.0) + EPS
        p = jnp.exp(y - m_ref[...])
        t_ref[...] = jnp.where(i >= nblk, p * y, p)

    return pl.pallas_call(
        body,
        grid=(2 * nblk,),
        in_specs=[
            pl.BlockSpec((BLK, HID), lambda i: (jax.lax.rem(i, nblk), 0)),
            pl.BlockSpec((1, HID), lambda i: (0, 0)),
        ],
        out_specs=pl.BlockSpec((BLK, HID), lambda i: (i, 0)),
        out_shape=jax.ShapeDtypeStruct((2 * n, HID), jnp.float32),
    )(x, M)


def _post(res, x_dst, W1, b1, W2, b2):
    n = x_dst.shape[0]
    nblk = n // BLK

    def body(den_ref, num_ref, x_ref, w1_ref, b1_ref, w2_ref, b2_ref,
             h2_ref, xn_ref):
        x = x_ref[...]
        g = num_ref[...] / (den_ref[...] + 1e-16) + x
        h = jnp.maximum(
            jax.lax.dot_general(g, w1_ref[...], (((1,), (0,)), ((), ())),
                                preferred_element_type=jnp.float32) + b1_ref[...], 0.0)
        h2 = jax.lax.dot_general(h, w2_ref[...], (((1,), (0,)), ((), ())),
                                 preferred_element_type=jnp.float32) + b2_ref[...]
        h2_ref[...] = h2
        xn_ref[...] = (jnp.maximum(h2, 0.0) + x) * 0.5

    return pl.pallas_call(
        body,
        grid=(nblk,),
        in_specs=[
            pl.BlockSpec((BLK, HID), lambda i: (i, 0)),
            pl.BlockSpec((BLK, HID), lambda i: (i + nblk, 0)),
            pl.BlockSpec((BLK, HID), lambda i: (i, 0)),
            pl.BlockSpec((HID, 2 * HID), lambda i: (0, 0)),
            pl.BlockSpec((1, 2 * HID), lambda i: (0, 0)),
            pl.BlockSpec((2 * HID, HID), lambda i: (0, 0)),
            pl.BlockSpec((1, HID), lambda i: (0, 0)),
        ],
        out_specs=[pl.BlockSpec((BLK, HID), lambda i: (i, 0)),
                   pl.BlockSpec((BLK, HID), lambda i: (i, 0))],
        out_shape=[jax.ShapeDtypeStruct((n, HID), jnp.float32),
                   jax.ShapeDtypeStruct((n, HID), jnp.float32)],
    )(res, res, x_dst, W1, b1.reshape(1, -1), W2, b2.reshape(1, -1))


LV = 40960
LC = 10240


def kernel(x_vals, x_cons, x_obj, pe_vals, pe_cons, pe_obj, src_c2v, dst_c2v, src_v2c, dst_v2c, params):
    p = params
    nv = x_vals.shape[0]
    nc = x_cons.shape[0]

    xvp = jnp.zeros((LV, x_vals.shape[1]), jnp.float32).at[:nv].set(x_vals)
    xcp = jnp.zeros((LC, x_cons.shape[1]), jnp.float32).at[:nc].set(x_cons)
    hv = _enc(xvp, p["W_enc_vals"], p["b_enc_vals"], p["b_pe_vals"])
    hc = _enc(xcp, p["W_enc_cons"], p["b_enc_cons"], p["b_pe_cons"])

    agg1_c2v = _sc_aggregate_write(nsrc=LC, nranges=5, rng=8192, seg=4736)
    agg1_v2c = _sc_aggregate_write(nsrc=LV, nranges=1, rng=10240, seg=2368)
    agg2_c2v = _sc_aggregate_read(nsrc=LC, nranges=5, rng=8192, seg=4736, nbuf=3)
    agg2_v2c = _sc_aggregate_read(nsrc=LV, nranges=1, rng=10240, seg=2368, nbuf=2)
    g_c2v, d_c2v = _pad_edges(src_c2v, dst_c2v, nv)
    g_v2c, d_v2c = _pad_edges(src_v2c, dst_v2c, nc)

    def mk_t(x_src):
        M = jnp.max(jnp.maximum(x_src, 0.0) + EPS, axis=0).reshape(1, HID)
        return _prep(x_src, M)

    res_v, lists_c2v, counts_c2v = agg1_c2v(mk_t(hc), g_c2v, d_c2v)
    res_c, lists_v2c, counts_v2c = agg1_v2c(mk_t(hv), g_v2c, d_v2c)
    h2_v1, hv = _post(res_v, hv, p["W1_c2v"], p["b1_c2v"], p["W2_c2v"], p["b2_c2v"])
    h2_c1, hc = _post(res_c, hc, p["W1_v2c"], p["b1_v2c"], p["W2_v2c"], p["b2_v2c"])

    res_v = agg2_c2v(mk_t(hc), lists_c2v, counts_c2v)
    res_c = agg2_v2c(mk_t(hv), lists_v2c, counts_v2c)
    h2_v2, _ = _post(res_v, hv, p["W1_c2v"], p["b1_c2v"], p["W2_c2v"], p["b2_c2v"])
    h2_c2, _ = _post(res_c, hc, p["W1_v2c"], p["b1_v2c"], p["W2_v2c"], p["b2_v2c"])

    ov = [_mlp(h, p["Wpv1"], p["bpv1"], p["Wpv2"], p["bpv2"])[:nv]
          for h in (h2_v1, h2_v2)]
    oc = [_mlp(h, p["Wpc1"], p["bpc1"], p["Wpc2"], p["bpc2"])[:nc, 0]
          for h in (h2_c1, h2_c2)]
    out_vals = jnp.stack(ov, axis=1)
    out_cons = jnp.stack(oc, axis=1)
    return (out_vals, out_cons)


# speedup vs baseline: 1.0429x; 1.0429x over previous
import functools

import jax
import jax.numpy as jnp
from jax import lax
from jax.experimental import pallas as pl
from jax.experimental.pallas import tpu as pltpu
from jax.experimental.pallas import tpu_sc as plsc

EPS = 1e-7
HID = 128

NTILES = 16
K = 128
EPT = 37888
EP = NTILES * EPT
ZROWS = 8


def _sc_aggregate(nsrc, nranges, rng, seg, nbuf=2):
    ar = rng + 2 * K
    nseg = EPT // seg
    nbmax = (seg + K + K - 1) // K
    zpt = ar // NTILES
    opt = rng // NTILES
    ngrp = (nbmax + nbuf - 1) // nbuf

    def body(t_hbm, src_hbm, dst_hbm, out_hbm, accum, segsrc, segdst, cg2, cd2, zbuf, *rest):
        rbufs = rest[:nbuf]
        sems = rest[nbuf:]
        c = lax.axis_index("c")
        s = lax.axis_index("s")
        bias = c * nsrc
        lanes = lax.iota(jnp.int32, 16)

        def zrow(i, _):
            def zcol(j, _):
                zbuf[i, pl.ds(j * 16, 16)] = jnp.zeros((16,), jnp.float32)
                return 0
            return lax.fori_loop(0, HID // 16, zcol, 0)
        lax.fori_loop(0, ZROWS, zrow, 0)

        for r in range(nranges):
            lo = r * rng
            def zacc(z, _):
                pltpu.sync_copy(zbuf, accum.at[pl.ds(s * zpt + z * ZROWS, ZROWS)])
                return 0
            lax.fori_loop(0, zpt // ZROWS, zacc, 0)
            plsc.subcore_barrier()

            def seg_body(sg, _):
                base = s * EPT + sg * seg
                pltpu.sync_copy(src_hbm.at[pl.ds(base, seg)], segsrc)
                pltpu.sync_copy(dst_hbm.at[pl.ds(base, seg)], segdst)

                def compact(i, n16):
                    src16 = segsrc[pl.ds(i * 16, 16)]
                    dst16 = segdst[pl.ds(i * 16, 16)]
                    msk = (dst16 >= lo) & (dst16 < lo + rng)
                    mi = msk.astype(jnp.int32)
                    pos = n16 + plsc.cumsum(mi) - mi
                    row = lax.shift_right_logical(pos, 7)
                    col = pos & (K - 1)
                    plsc.store_scatter(cg2, [row, col], src16 + bias, mask=msk)
                    plsc.store_scatter(cd2, [row, col], dst16 - lo, mask=msk)
                    return n16 + plsc.all_reduce_population_count(msk)
                n16 = lax.fori_loop(0, seg // 16, compact,
                                    jnp.zeros((16,), jnp.int32))
                for t in range(K // 16):
                    pos = n16 + lanes + t * 16
                    row = lax.shift_right_logical(pos, 7)
                    col = pos & (K - 1)
                    plsc.store_scatter(cg2, [row, col], bias + col)
                    plsc.store_scatter(cd2, [row, col], rng + col)
                n_sc = jnp.max(n16)
                nb = lax.shift_right_logical(n_sc + (K - 1), 7)

                for j in range(nbuf):
                    @pl.when(j < nb)
                    def _():
                        pltpu.async_copy(t_hbm.at[cg2.at[j]], rbufs[j], sems[j])

                def grp(g, _):
                    for j in range(nbuf):
                        b = g * nbuf + j

                        @pl.when(b < nb)
                        def _():
                            pltpu.make_async_copy(
                                t_hbm.at[cg2.at[b]], rbufs[j], sems[j]).wait()
                            pltpu.sync_copy(rbufs[j], accum.at[cd2.at[b]], add=True)

                            @pl.when(b + nbuf < nb)
                            def _():
                                pltpu.async_copy(
                                    t_hbm.at[cg2.at[b + nbuf]], rbufs[j], sems[j])
                    return 0
                lax.fori_loop(0, ngrp, grp, 0)
                return 0
            lax.fori_loop(0, nseg, seg_body, 0)
            plsc.subcore_barrier()

            pltpu.sync_copy(accum.at[pl.ds(s * opt, opt)],
                            out_hbm.at[pl.ds((c * nranges + r) * rng + s * opt, opt)])
            plsc.subcore_barrier()

    mesh = plsc.VectorSubcoreMesh(core_axis_name="c", subcore_axis_name="s")
    return pl.kernel(
        body,
        out_type=jax.ShapeDtypeStruct((2 * nranges * rng, HID), jnp.float32),
        mesh=mesh,
        compiler_params=pltpu.CompilerParams(needs_layout_passes=False),
        scratch_types=(
            [pltpu.VMEM_SHARED((ar, HID), jnp.float32),
             pltpu.VMEM((seg,), jnp.int32),
             pltpu.VMEM((seg,), jnp.int32),
             pltpu.VMEM((nbmax + 2, K), jnp.int32),
             pltpu.VMEM((nbmax + 2, K), jnp.int32),
             pltpu.VMEM((ZROWS, HID), jnp.float32)]
            + [pltpu.VMEM((K, HID), jnp.float32) for _ in range(nbuf)]
            + [pltpu.SemaphoreType.DMA for _ in range(nbuf)]
        ),
    )


def _pad_edges(src, dst, ndst):
    npad = EP - src.shape[0]
    fill = jnp.arange(npad, dtype=jnp.int32) % K
    gsrc = jnp.concatenate([src.astype(jnp.int32), fill])
    dpad = jnp.concatenate([dst.astype(jnp.int32), ndst + fill])
    return gsrc, dpad


def _mlp_body(g_ref, w1_ref, b1_ref, w2_ref, b2_ref, o_ref):
    g = g_ref[...]
    h = jnp.maximum(
        jax.lax.dot_general(g, w1_ref[...], (((1,), (0,)), ((), ())),
                            preferred_element_type=jnp.float32) + b1_ref[...], 0.0)
    o_ref[...] = jax.lax.dot_general(h, w2_ref[...], (((1,), (0,)), ((), ())),
                                     preferred_element_type=jnp.float32) + b2_ref[...]


BLK = 1024


def _mlp(g, W1, b1, W2, b2):
    n, k = g.shape
    k2 = W1.shape[1]
    ko = W2.shape[1]
    return pl.pallas_call(
        _mlp_body,
        grid=(n // BLK,),
        in_specs=[
            pl.BlockSpec((BLK, k), lambda i: (i, 0)),
            pl.BlockSpec((k, k2), lambda i: (0, 0)),
            pl.BlockSpec((1, k2), lambda i: (0, 0)),
            pl.BlockSpec((k2, ko), lambda i: (0, 0)),
            pl.BlockSpec((1, ko), lambda i: (0, 0)),
        ],
        out_specs=pl.BlockSpec((BLK, ko), lambda i: (i, 0)),
        out_shape=jax.ShapeDtypeStruct((n, ko), jnp.float32),
    )(g, W1, b1.reshape(1, -1), W2, b2.reshape(1, -1))


def _enc(xp, We, be, bp):
    n = xp.shape[0]
    fin = xp.shape[1]
    h2 = We.shape[1]

    def body(x_ref, we_ref, be_ref, bp_ref, o_ref):
        a = jax.lax.dot_general(x_ref[...], we_ref[...], (((1,), (0,)), ((), ())),
                                preferred_element_type=jnp.float32) + be_ref[...]
        b = jnp.broadcast_to(bp_ref[...], (BLK, h2))
        o_ref[...] = jnp.maximum(jnp.concatenate([a, b], axis=1), 0.0)

    return pl.pallas_call(
        body,
        grid=(n // BLK,),
        in_specs=[
            pl.BlockSpec((BLK, fin), lambda i: (i, 0)),
            pl.BlockSpec((fin, h2), lambda i: (0, 0)),
            pl.BlockSpec((1, h2), lambda i: (0, 0)),
            pl.BlockSpec((1, h2), lambda i: (0, 0)),
        ],
        out_specs=pl.BlockSpec((BLK, 2 * h2), lambda i: (i, 0)),
        out_shape=jax.ShapeDtypeStruct((n, 2 * h2), jnp.float32),
    )(xp, We, be.reshape(1, -1), bp.reshape(1, -1))


def _prep(x, M):
    n = x.shape[0]
    nblk = n // BLK

    def body(x_ref, m_ref, t_ref):
        i = pl.program_id(0)
        y = jnp.maximum(x_ref[...], 0.0) + EPS
        p = jnp.exp(y - m_ref[...])
        t_ref[...] = jnp.where(i >= nblk, p * y, p)

    return pl.pallas_call(
        body,
        grid=(2 * nblk,),
        in_specs=[
            pl.BlockSpec((BLK, HID), lambda i: (jax.lax.rem(i, nblk), 0)),
            pl.BlockSpec((1, HID), lambda i: (0, 0)),
        ],
        out_specs=pl.BlockSpec((BLK, HID), lambda i: (i, 0)),
        out_shape=jax.ShapeDtypeStruct((2 * n, HID), jnp.float32),
    )(x, M)


def _post(res, x_dst, W1, b1, W2, b2):
    n = x_dst.shape[0]
    nblk = n // BLK

    def body(den_ref, num_ref, x_ref, w1_ref, b1_ref, w2_ref, b2_ref,
             h2_ref, xn_ref):
        x = x_ref[...]
        g = num_ref[...] / (den_ref[...] + 1e-16) + x
        h = jnp.maximum(
            jax.lax.dot_general(g, w1_ref[...], (((1,), (0,)), ((), ())),
                                preferred_element_type=jnp.float32) + b1_ref[...], 0.0)
        h2 = jax.lax.dot_general(h, w2_ref[...], (((1,), (0,)), ((), ())),
                                 preferred_element_type=jnp.float32) + b2_ref[...]
        h2_ref[...] = h2
        xn_ref[...] = (jnp.maximum(h2, 0.0) + x) * 0.5

    return pl.pallas_call(
        body,
        grid=(nblk,),
        in_specs=[
            pl.BlockSpec((BLK, HID), lambda i: (i, 0)),
            pl.BlockSpec((BLK, HID), lambda i: (i + nblk, 0)),
            pl.BlockSpec((BLK, HID), lambda i: (i, 0)),
            pl.BlockSpec((HID, 2 * HID), lambda i: (0, 0)),
            pl.BlockSpec((1, 2 * HID), lambda i: (0, 0)),
            pl.BlockSpec((2 * HID, HID), lambda i: (0, 0)),
            pl.BlockSpec((1, HID), lambda i: (0, 0)),
        ],
        out_specs=[pl.BlockSpec((BLK, HID), lambda i: (i, 0)),
                   pl.BlockSpec((BLK, HID), lambda i: (i, 0))],
        out_shape=[jax.ShapeDtypeStruct((n, HID), jnp.float32),
                   jax.ShapeDtypeStruct((n, HID), jnp.float32)],
    )(res, res, x_dst, W1, b1.reshape(1, -1), W2, b2.reshape(1, -1))


LV = 40960
LC = 10240


def kernel(x_vals, x_cons, x_obj, pe_vals, pe_cons, pe_obj, src_c2v, dst_c2v, src_v2c, dst_v2c, params):
    p = params
    nv = x_vals.shape[0]
    nc = x_cons.shape[0]

    xvp = jnp.zeros((LV, x_vals.shape[1]), jnp.float32).at[:nv].set(x_vals)
    xcp = jnp.zeros((LC, x_cons.shape[1]), jnp.float32).at[:nc].set(x_cons)
    hv = _enc(xvp, p["W_enc_vals"], p["b_enc_vals"], p["b_pe_vals"])
    hc = _enc(xcp, p["W_enc_cons"], p["b_enc_cons"], p["b_pe_cons"])

    agg_c2v = _sc_aggregate(nsrc=LC, nranges=5, rng=8192, seg=4736)
    agg_v2c = _sc_aggregate(nsrc=LV, nranges=1, rng=10240, seg=2368)
    g_c2v, d_c2v = _pad_edges(src_c2v, dst_c2v, nv)
    g_v2c, d_v2c = _pad_edges(src_v2c, dst_v2c, nc)

    def conv(x_src, x_dst, gsrc, dpad, agg, W1, b1, W2, b2):
        M = jnp.max(jnp.maximum(x_src, 0.0) + EPS, axis=0).reshape(1, HID)
        T = _prep(x_src, M)
        res = agg(T, gsrc, dpad)
        return _post(res, x_dst, W1, b1, W2, b2)

    hid_c, hid_v = [], []
    for _ in range(2):
        h2_v, hv_n = conv(hc, hv, g_c2v, d_c2v, agg_c2v,
                          p["W1_c2v"], p["b1_c2v"], p["W2_c2v"], p["b2_c2v"])
        h2_c, hc_n = conv(hv, hc, g_v2c, d_v2c, agg_v2c,
                          p["W1_v2c"], p["b1_v2c"], p["W2_v2c"], p["b2_v2c"])
        hid_v.append(h2_v)
        hid_c.append(h2_c)
        hv, hc = hv_n, hc_n

    ov = [_mlp(h, p["Wpv1"], p["bpv1"], p["Wpv2"], p["bpv2"])[:nv] for h in hid_v]
    oc = [_mlp(h, p["Wpc1"], p["bpc1"], p["Wpc2"], p["bpc2"])[:nc, 0] for h in hid_c]
    out_vals = jnp.stack(ov, axis=1)
    out_cons = jnp.stack(oc, axis=1)
    return (out_vals, out_cons)
